# baseline (speedup 1.0000x reference)
import functools

import jax
import jax.numpy as jnp
from jax.experimental import pallas as pl
from jax.experimental.pallas import tpu as pltpu

_A = 9
_G = 16
_NK = 16


def _rpn_body(H, W, x_ref, w1_ref, b1_ref, w2_ref, b2_ref,
              wreg_ref, breg_ref, wcls_ref, bcls_ref, out_ref,
              xb_ref, yb_ref):
    C = b1_ref.shape[1]
    A = _A
    XQ = (H + 2) * W
    NP = H * W
    MC = NP // _NK
    zero = jnp.zeros((), dtype=jnp.bfloat16)

    def zero_bands(buf):
        buf[pl.ds(0, _G + W), C:2 * C] = jnp.zeros((_G + W, C),
                                                   dtype=jnp.bfloat16)
        buf[pl.ds(_G + W + NP, _G + W), C:2 * C] = jnp.zeros(
            (_G + W, C), dtype=jnp.bfloat16)

    def stage(k):
        xb_ref[pl.ds(_G + W + k * MC, MC), C:2 * C] = \
            x_ref[0, pl.ds(k * MC, MC), :].astype(jnp.bfloat16)

    def taps(buf, k):
        t0 = _G if k == 0 else _G + 2 * W + k * MC
        ln = MC + 2 * W if k == 0 else MC
        wpos = jax.lax.broadcasted_iota(jnp.int32, (ln, 1), 0) + (t0 - _G)
        wpos = wpos % W
        left = buf[pl.ds(t0 - 1, ln), C:2 * C]
        buf[pl.ds(t0, ln), 0:C] = jnp.where(wpos == 0, zero, left)
        right = buf[pl.ds(t0 + 1, ln), C:2 * C]
        buf[pl.ds(t0, ln), 2 * C:3 * C] = jnp.where(wpos == W - 1, zero,
                                                    right)

    def conv_chunk(buf, w, b_ref, k):
        acc = jnp.dot(buf[pl.ds(_G + k * MC, MC), :], w[0],
                      preferred_element_type=jnp.float32)
        acc += jnp.dot(buf[pl.ds(_G + W + k * MC, MC), :], w[1],
                       preferred_element_type=jnp.float32)
        acc += jnp.dot(buf[pl.ds(_G + 2 * W + k * MC, MC), :], w[2],
                       preferred_element_type=jnp.float32)
        return jnp.maximum(acc + b_ref[0], 0.0).astype(jnp.bfloat16)

    def head(k, h2k):
        z = jnp.dot(h2k, wcat, preferred_element_type=jnp.float32) + bcat
        ch = jax.lax.broadcasted_iota(jnp.int32, (1, 6 * A), 1) % 6
        out_ref[0, pl.ds(k * MC, MC), :] = jnp.where(
            ch < 2, 1.0 / (1.0 + jnp.exp(-z)), z)

    w1c = w1_ref[...].astype(jnp.bfloat16)
    w2c = w2_ref[...].astype(jnp.bfloat16)
    K6 = 6 * A
    r = jax.lax.broadcasted_iota(jnp.int32, (K6, K6), 0)
    c = jax.lax.broadcasted_iota(jnp.int32, (K6, K6), 1)
    a6 = c // 6
    j6 = c % 6
    plus = ((j6 >= 2) & (r == 2 * A + 4 * a6 + j6 - 2)) \
        | ((j6 == 0) & (r == 2 * a6)) | ((j6 == 1) & (r == 2 * a6 + 1))
    minus = ((j6 == 0) & (r == 2 * a6 + 1)) | ((j6 == 1) & (r == 2 * a6))
    P = jnp.where(plus, 1.0, 0.0) - jnp.where(minus, 1.0, 0.0)
    whead = jnp.concatenate([wcls_ref[...], wreg_ref[...]], axis=1)
    wcat = jnp.dot(whead, P,
                   preferred_element_type=jnp.float32).astype(jnp.bfloat16)
    bvec = jnp.concatenate([bcls_ref[...], breg_ref[...]], axis=1)
    bcat = jnp.dot(bvec, P, preferred_element_type=jnp.float32)

    zero_bands(xb_ref)
    zero_bands(yb_ref)

    def c1(k):
        yb_ref[pl.ds(_G + W + k * MC, MC), C:2 * C] = \
            conv_chunk(xb_ref, w1c, b1_ref, k)

    def c2h(k):
        head(k, conv_chunk(yb_ref, w2c, b2_ref, k))

    stage(0)
    stage(1)
    for k in range(_NK):
        if 1 <= k and k + 1 < _NK:
            stage(k + 1)
        taps(xb_ref, k)
        c1(k)
        if k >= 1:
            taps(yb_ref, k - 1)
        if k >= 2:
            c2h(k - 2)
    taps(yb_ref, _NK - 1)
    c2h(_NK - 2)
    c2h(_NK - 1)


def kernel(input, W1, b1, W2, b2, Wreg, breg, Wcls, bcls):
    B, H, W, C = input.shape
    A = _A
    NP = H * W
    TOT = (H + 2) * W + 2 * _G
    MC = NP // _NK

    x2d = input.reshape(B, NP, C)
    W1r = W1.reshape(3, 3 * C, C)
    W2r = W2.reshape(3, 3 * C, C)

    out = pl.pallas_call(
        functools.partial(_rpn_body, H, W),
        grid=(B,),
        in_specs=[
            pl.BlockSpec((1, NP, C), lambda b: (b, 0, 0)),
            pl.BlockSpec((3, 3 * C, C), lambda b: (0, 0, 0)),
            pl.BlockSpec((1, C), lambda b: (0, 0)),
            pl.BlockSpec((3, 3 * C, C), lambda b: (0, 0, 0)),
            pl.BlockSpec((1, C), lambda b: (0, 0)),
            pl.BlockSpec((C, 4 * A), lambda b: (0, 0)),
            pl.BlockSpec((1, 4 * A), lambda b: (0, 0)),
            pl.BlockSpec((C, 2 * A), lambda b: (0, 0)),
            pl.BlockSpec((1, 2 * A), lambda b: (0, 0)),
        ],
        out_specs=pl.BlockSpec((1, NP, 6 * A), lambda b: (b, 0, 0)),
        out_shape=jax.ShapeDtypeStruct((B, NP, 6 * A), jnp.float32),
        scratch_shapes=[
            pltpu.VMEM((TOT, 3 * C), jnp.bfloat16),
            pltpu.VMEM((TOT, 3 * C), jnp.bfloat16),
        ],
    )(x2d, W1r, b1.reshape(1, C), W2r, b2.reshape(1, C),
      Wreg.reshape(C, 4 * A), breg.reshape(1, 4 * A),
      Wcls.reshape(C, 2 * A), bcls.reshape(1, 2 * A))

    return out.reshape(B, H, W, A, 6)


# speedup vs baseline: 1.0328x; 1.0328x over previous
import functools

import jax
import jax.numpy as jnp
from jax.experimental import pallas as pl
from jax.experimental.pallas import tpu as pltpu

_A = 9
_G = 16
_NK = 8


def _rpn_body(H, W, x_ref, w1_ref, b1_ref, w2_ref, b2_ref,
              wreg_ref, breg_ref, wcls_ref, bcls_ref, out_ref,
              xb_ref, yb_ref):
    C = b1_ref.shape[1]
    A = _A
    XQ = (H + 2) * W
    NP = H * W
    MC = NP // _NK
    zero = jnp.zeros((), dtype=jnp.bfloat16)

    def zero_bands(buf):
        buf[pl.ds(0, _G + W), C:2 * C] = jnp.zeros((_G + W, C),
                                                   dtype=jnp.bfloat16)
        buf[pl.ds(_G + W + NP, _G + W), C:2 * C] = jnp.zeros(
            (_G + W, C), dtype=jnp.bfloat16)

    def stage(k):
        xb_ref[pl.ds(_G + W + k * MC, MC), C:2 * C] = \
            x_ref[0, pl.ds(k * MC, MC), :].astype(jnp.bfloat16)

    def taps(buf, k):
        t0 = _G if k == 0 else _G + 2 * W + k * MC
        ln = MC + 2 * W if k == 0 else MC
        wpos = jax.lax.broadcasted_iota(jnp.int32, (ln, 1), 0) + (t0 - _G)
        wpos = wpos % W
        left = buf[pl.ds(t0 - 1, ln), C:2 * C]
        buf[pl.ds(t0, ln), 0:C] = jnp.where(wpos == 0, zero, left)
        right = buf[pl.ds(t0 + 1, ln), C:2 * C]
        buf[pl.ds(t0, ln), 2 * C:3 * C] = jnp.where(wpos == W - 1, zero,
                                                    right)

    def conv_chunk(buf, w, b_ref, k):
        acc = jnp.dot(buf[pl.ds(_G + k * MC, MC), :], w[0],
                      preferred_element_type=jnp.float32)
        acc += jnp.dot(buf[pl.ds(_G + W + k * MC, MC), :], w[1],
                       preferred_element_type=jnp.float32)
        acc += jnp.dot(buf[pl.ds(_G + 2 * W + k * MC, MC), :], w[2],
                       preferred_element_type=jnp.float32)
        return jnp.maximum(acc + b_ref[0], 0.0).astype(jnp.bfloat16)

    def head(k, h2k):
        z = jnp.dot(h2k, wcat, preferred_element_type=jnp.float32) + bcat
        ch = jax.lax.broadcasted_iota(jnp.int32, (1, 6 * A), 1) % 6
        out_ref[0, pl.ds(k * MC, MC), :] = jnp.where(
            ch < 2, 1.0 / (1.0 + jnp.exp(-z)), z)

    w1c = w1_ref[...].astype(jnp.bfloat16)
    w2c = w2_ref[...].astype(jnp.bfloat16)
    K6 = 6 * A
    r = jax.lax.broadcasted_iota(jnp.int32, (K6, K6), 0)
    c = jax.lax.broadcasted_iota(jnp.int32, (K6, K6), 1)
    a6 = c // 6
    j6 = c % 6
    plus = ((j6 >= 2) & (r == 2 * A + 4 * a6 + j6 - 2)) \
        | ((j6 == 0) & (r == 2 * a6)) | ((j6 == 1) & (r == 2 * a6 + 1))
    minus = ((j6 == 0) & (r == 2 * a6 + 1)) | ((j6 == 1) & (r == 2 * a6))
    P = jnp.where(plus, 1.0, 0.0) - jnp.where(minus, 1.0, 0.0)
    whead = jnp.concatenate([wcls_ref[...], wreg_ref[...]], axis=1)
    wcat = jnp.dot(whead, P,
                   preferred_element_type=jnp.float32).astype(jnp.bfloat16)
    bvec = jnp.concatenate([bcls_ref[...], breg_ref[...]], axis=1)
    bcat = jnp.dot(bvec, P, preferred_element_type=jnp.float32)

    zero_bands(xb_ref)
    zero_bands(yb_ref)

    def c1(k):
        yb_ref[pl.ds(_G + W + k * MC, MC), C:2 * C] = \
            conv_chunk(xb_ref, w1c, b1_ref, k)

    def c2h(k):
        head(k, conv_chunk(yb_ref, w2c, b2_ref, k))

    stage(0)
    stage(1)
    for k in range(_NK):
        if 1 <= k and k + 1 < _NK:
            stage(k + 1)
        taps(xb_ref, k)
        c1(k)
        if k >= 1:
            taps(yb_ref, k - 1)
        if k >= 2:
            c2h(k - 2)
    taps(yb_ref, _NK - 1)
    c2h(_NK - 2)
    c2h(_NK - 1)


def kernel(input, W1, b1, W2, b2, Wreg, breg, Wcls, bcls):
    B, H, W, C = input.shape
    A = _A
    NP = H * W
    TOT = (H + 2) * W + 2 * _G
    MC = NP // _NK

    x2d = input.reshape(B, NP, C)
    W1r = W1.reshape(3, 3 * C, C)
    W2r = W2.reshape(3, 3 * C, C)

    out = pl.pallas_call(
        functools.partial(_rpn_body, H, W),
        grid=(B,),
        in_specs=[
            pl.BlockSpec((1, NP, C), lambda b: (b, 0, 0)),
            pl.BlockSpec((3, 3 * C, C), lambda b: (0, 0, 0)),
            pl.BlockSpec((1, C), lambda b: (0, 0)),
            pl.BlockSpec((3, 3 * C, C), lambda b: (0, 0, 0)),
            pl.BlockSpec((1, C), lambda b: (0, 0)),
            pl.BlockSpec((C, 4 * A), lambda b: (0, 0)),
            pl.BlockSpec((1, 4 * A), lambda b: (0, 0)),
            pl.BlockSpec((C, 2 * A), lambda b: (0, 0)),
            pl.BlockSpec((1, 2 * A), lambda b: (0, 0)),
        ],
        out_specs=pl.BlockSpec((1, NP, 6 * A), lambda b: (b, 0, 0)),
        out_shape=jax.ShapeDtypeStruct((B, NP, 6 * A), jnp.float32),
        scratch_shapes=[
            pltpu.VMEM((TOT, 3 * C), jnp.bfloat16),
            pltpu.VMEM((TOT, 3 * C), jnp.bfloat16),
        ],
    )(x2d, W1r, b1.reshape(1, C), W2r, b2.reshape(1, C),
      Wreg.reshape(C, 4 * A), breg.reshape(1, 4 * A),
      Wcls.reshape(C, 2 * A), bcls.reshape(1, 2 * A))

    return out.reshape(B, H, W, A, 6)
